# baseline (speedup 1.0000x reference)
import functools

import jax
import jax.numpy as jnp
from jax.experimental import pallas as pl

S = 16384
H = 2
BLK = 256
F32 = jnp.float32


def _embed_refine(x, eW1, eb1, eW2, eb2, rW1, rb1, rW2, rb2):
    h1 = jnp.maximum(x * eW1 + eb1, 0.0)
    emb = jnp.dot(h1, eW2, preferred_element_type=F32) + eb2
    h2 = jnp.maximum(jnp.dot(emb, rW1, preferred_element_type=F32) + rb1, 0.0)
    ref = jnp.dot(h2, rW2, preferred_element_type=F32) + rb2
    return emb, ref


def _shift(ref, amax_row):
    s = jnp.sum(jnp.abs(ref) * amax_row, axis=1, keepdims=True)
    return jnp.minimum(s, 44.0)


def _write_body(x_ref, y_ref, eW1, eb1, eW2, eb2, rW1, rb1, rW2, rb2,
                amax_ref, a_ref, minit_ref, out_ref):
    i = pl.program_id(0)

    @pl.when(i == 0)
    def _init():
        out_ref[:] = minit_ref[:]

    emb, ref = _embed_refine(x_ref[:], eW1[:], eb1[:], eW2[:], eb2[:],
                             rW1[:], rb1[:], rW2[:], rb2[:])
    y = y_ref[:]
    val = jnp.concatenate([emb * y, y], axis=1)
    s = _shift(ref, amax_ref[:])
    for h in range(H):
        a_h = a_ref[:, h * S:(h + 1) * S]
        logits = jnp.dot(ref, a_h, preferred_element_type=F32)
        e = jnp.exp(logits - s)
        denom = jnp.sum(e, axis=1, keepdims=True)
        valh = val / denom
        dmt = jax.lax.dot_general(valh, e, (((0,), (0,)), ((), ())),
                                  preferred_element_type=F32)
        out_ref[h * 24:(h + 1) * 24, :] += dmt


NCH = 2
CH = S // NCH


def _read_body(x_ref, ws_ref, eW1, eb1, eW2, eb2, rW1, rb1, rW2, rb2,
               amax_ref, at_ref, mtn_ref, w1h0, w1h1, w1sq0, w1sq1, w1e, w1w,
               b1eff, dW2, db2, dW3, db3, out_ref):
    emb, ref = _embed_refine(x_ref[:], eW1[:], eb1[:], eW2[:], eb2[:],
                             rW1[:], rb1[:], rW2[:], rb2[:])
    acc = (jnp.dot(emb, w1e[:], preferred_element_type=F32)
           + ws_ref[:] * w1w[:] + b1eff[:])
    refT = ref.T
    z = jnp.zeros_like(refT)
    refpad = jnp.concatenate(
        [jnp.concatenate([refT, z], axis=1),
         jnp.concatenate([z, refT], axis=1)], axis=0)
    s_row = jnp.minimum(jnp.sum(jnp.abs(refT) * amax_ref[:], axis=0,
                                keepdims=True), 44.0)
    s_cat = jnp.concatenate([s_row, s_row], axis=1)
    read_un = jnp.zeros((2 * BLK, 64), F32)
    e2row = jnp.zeros((1, 2 * BLK), F32)
    for c in range(NCH):
        at_c = at_ref[c * CH:(c + 1) * CH, :]
        lt = jnp.dot(at_c, refpad, preferred_element_type=F32)
        et = jnp.exp(lt - s_cat)
        e2row = e2row + jnp.sum(et * et, axis=0, keepdims=True)
        read_un = read_un + jax.lax.dot_general(
            et, mtn_ref[c * CH:(c + 1) * CH, :],
            (((0,), (0,)), ((), ())), preferred_element_type=F32)
    e2col = jnp.reshape(e2row, (2 * BLK, 1))
    w1h = (w1h0, w1h1)
    w1sq = (w1sq0, w1sq1)
    for h in range(H):
        ru = read_un[h * BLK:(h + 1) * BLK, h * 32:h * 32 + 32]
        denom = ru[:, 24:25]
        read = ru[:, 0:24] / denom
        asq = e2col[h * BLK:(h + 1) * BLK, :] / (denom * denom)
        acc = acc + jnp.dot(read, w1h[h][:], preferred_element_type=F32)
        acc = acc + asq * w1sq[h][:]
    h1 = jnp.maximum(acc, 0.0)
    hh = h1 + jnp.maximum(jnp.dot(h1, dW2[:], preferred_element_type=F32)
                          + db2[:], 0.0)
    out_ref[:] = jnp.dot(hh, dW3[:], preferred_element_type=F32) + db3[:]


def _full(shape):
    n = len(shape)
    return pl.BlockSpec(shape, lambda i, _n=n: (0,) * _n)


@functools.partial(jax.jit, static_argnums=())
def kernel(input_x, input_y, query_x, weight_sum_tensor,
           emb_W1, emb_b1, emb_W2, emb_b2,
           ref_W1, ref_b1, ref_W2, ref_b2,
           attn_A, mem_M, mem_C,
           dec_W1, dec_b1, dec_W2, dec_b2, dec_W3, dec_b3):
    B = input_x.shape[0]
    Q = query_x.shape[0]

    eb1 = emb_b1.reshape(1, -1)
    eb2 = emb_b2.reshape(1, -1)
    rb1 = ref_b1.reshape(1, -1)
    rb2 = ref_b2.reshape(1, -1)
    a_cat = jnp.concatenate([attn_A[0], attn_A[1]], axis=1)
    amax = jnp.max(jnp.abs(attn_A), axis=(0, 2)).reshape(1, 5)
    minit = jnp.concatenate([mem_M[0].T, mem_C[0][None, :],
                             mem_M[1].T, mem_C[1][None, :]], axis=0)
    w1h0 = jnp.concatenate([dec_W1[0:23], dec_W1[46:47]], axis=0)
    w1h1 = jnp.concatenate([dec_W1[23:46], dec_W1[47:48]], axis=0)
    w1sq0 = dec_W1[50:51]
    w1sq1 = dec_W1[51:52]
    w1e = dec_W1[52:75]
    w1w = dec_W1[75:76]
    b1eff = (dec_b1 + dec_W1[48] + dec_W1[49]).reshape(1, -1)
    db2 = dec_b2.reshape(1, -1)
    db3 = dec_b3.reshape(1, -1)

    row_spec = pl.BlockSpec((BLK, 1), lambda i: (i, 0))
    wspecs = [_full((1, 64)), _full((1, 64)), _full((64, 23)), _full((1, 23)),
              _full((23, 32)), _full((1, 32)), _full((32, 5)), _full((1, 5))]

    mt = pl.pallas_call(
        _write_body,
        grid=(B // BLK,),
        in_specs=[row_spec, row_spec, *wspecs, _full((1, 5)),
                  _full((5, H * S)), _full((H * 24, S))],
        out_specs=_full((H * 24, S)),
        out_shape=jax.ShapeDtypeStruct((H * 24, S), F32),
    )(input_x, input_y, emb_W1.reshape(1, -1), eb1, emb_W2, eb2,
      ref_W1, rb1, ref_W2, rb2, amax, a_cat, minit)

    ones_col = jnp.ones((S, 1), F32)
    zpad = jnp.zeros((S, 7), F32)
    mtn = jnp.concatenate([mt[0:24].T, ones_col, zpad,
                           mt[24:48].T, ones_col, zpad], axis=1)
    at10 = jnp.concatenate([attn_A[0].T, attn_A[1].T], axis=1)

    dec_pred = pl.pallas_call(
        _read_body,
        grid=(Q // BLK,),
        in_specs=[row_spec, row_spec, *wspecs, _full((5, 1)),
                  _full((S, 10)), _full((S, 64)),
                  _full((24, 256)), _full((24, 256)),
                  _full((1, 256)), _full((1, 256)),
                  _full((23, 256)), _full((1, 256)), _full((1, 256)),
                  _full((256, 256)), _full((1, 256)),
                  _full((256, 1)), _full((1, 1))],
        out_specs=row_spec,
        out_shape=jax.ShapeDtypeStruct((Q, 1), F32),
    )(query_x, weight_sum_tensor, emb_W1.reshape(1, -1), eb1, emb_W2, eb2,
      ref_W1, rb1, ref_W2, rb2, amax.reshape(5, 1), at10, mtn,
      w1h0, w1h1, w1sq0, w1sq1, w1e, w1w, b1eff, dec_W2, db2, dec_W3, db3)

    return dec_pred


# speedup vs baseline: 1.0609x; 1.0609x over previous
import functools

import jax
import jax.numpy as jnp
from jax.experimental import pallas as pl
from jax.experimental.pallas import tpu as pltpu

S = 16384
H = 2
BLK = 256
NB = 4096 // BLK
F32 = jnp.float32


def _embed_refine(x, eW1, eb1, eW2, eb2, rW1, rb1, rW2, rb2):
    h1 = jnp.maximum(x * eW1 + eb1, 0.0)
    emb = jnp.dot(h1, eW2, preferred_element_type=F32) + eb2
    h2 = jnp.maximum(jnp.dot(emb, rW1, preferred_element_type=F32) + rb1, 0.0)
    ref = jnp.dot(h2, rW2, preferred_element_type=F32) + rb2
    return emb, ref


def _shift(ref, amax_row):
    s = jnp.sum(jnp.abs(ref) * amax_row, axis=1, keepdims=True)
    return jnp.minimum(s, 44.0)


def _body(x_ref, y_ref, qx_ref, ws_ref, eW1, eb1, eW2, eb2, rW1, rb1,
          rW2, rb2, amax_ref, a_ref, minit_ref, w1h0, w1h1, w1sq0, w1sq1,
          w1e, w1w, b1eff, dW2, db2, dW3, db3, out_ref, macc_ref):
    i = pl.program_id(0)

    @pl.when(i == 0)
    def _init():
        macc_ref[:] = minit_ref[:]

    @pl.when(i < NB)
    def _write():
        emb, ref = _embed_refine(x_ref[:], eW1[:], eb1[:], eW2[:], eb2[:],
                                 rW1[:], rb1[:], rW2[:], rb2[:])
        y = y_ref[:]
        val = jnp.concatenate([emb * y, y], axis=1)
        s = _shift(ref, amax_ref[:])
        for h in range(H):
            a_h = a_ref[:, h * S:(h + 1) * S]
            logits = jnp.dot(ref, a_h, preferred_element_type=F32)
            e = jnp.exp(logits - s)
            denom = jnp.sum(e, axis=1, keepdims=True)
            valh = val / denom
            dmt = jax.lax.dot_general(valh, e, (((0,), (0,)), ((), ())),
                                      preferred_element_type=F32)
            macc_ref[h * 32:h * 32 + 24, :] += dmt

    @pl.when(i >= NB)
    def _read():
        emb, ref = _embed_refine(qx_ref[:], eW1[:], eb1[:], eW2[:], eb2[:],
                                 rW1[:], rb1[:], rW2[:], rb2[:])
        acc = (jnp.dot(emb, w1e[:], preferred_element_type=F32)
               + ws_ref[:] * w1w[:] + b1eff[:])
        w1h = (w1h0, w1h1)
        w1sq = (w1sq0, w1sq1)
        s = _shift(ref, amax_ref[:])
        for h in range(H):
            a_h = a_ref[:, h * S:(h + 1) * S]
            logits = jnp.dot(ref, a_h, preferred_element_type=F32)
            e = jnp.exp(logits - s)
            e2 = jnp.sum(e * e, axis=1, keepdims=True)
            read_un = jax.lax.dot_general(
                e, macc_ref[h * 32:(h + 1) * 32, :],
                (((1,), (1,)), ((), ())),
                preferred_element_type=F32)
            denom = read_un[:, 24:25]
            read = read_un[:, 0:24] / denom
            asq = e2 / (denom * denom)
            acc = acc + jnp.dot(read, w1h[h][:], preferred_element_type=F32)
            acc = acc + asq * w1sq[h][:]
        h1 = jnp.maximum(acc, 0.0)
        hh = h1 + jnp.maximum(jnp.dot(h1, dW2[:], preferred_element_type=F32)
                              + db2[:], 0.0)
        out_ref[:] = jnp.dot(hh, dW3[:], preferred_element_type=F32) + db3[:]


def _full(shape):
    n = len(shape)
    return pl.BlockSpec(shape, lambda i, _n=n: (0,) * _n)


@functools.partial(jax.jit, static_argnums=())
def kernel(input_x, input_y, query_x, weight_sum_tensor,
           emb_W1, emb_b1, emb_W2, emb_b2,
           ref_W1, ref_b1, ref_W2, ref_b2,
           attn_A, mem_M, mem_C,
           dec_W1, dec_b1, dec_W2, dec_b2, dec_W3, dec_b3):
    Q = query_x.shape[0]

    eb1 = emb_b1.reshape(1, -1)
    eb2 = emb_b2.reshape(1, -1)
    rb1 = ref_b1.reshape(1, -1)
    rb2 = ref_b2.reshape(1, -1)
    a_cat = jnp.concatenate([attn_A[0], attn_A[1]], axis=1)
    amax = jnp.max(jnp.abs(attn_A), axis=(0, 2)).reshape(1, 5)
    ones_row = jnp.ones((1, S), F32)
    zpad = jnp.zeros((7, S), F32)
    minit = jnp.concatenate([mem_M[0].T, mem_C[0][None, :], ones_row, zpad,
                             mem_M[1].T, mem_C[1][None, :], ones_row, zpad],
                            axis=0)
    w1h0 = jnp.concatenate([dec_W1[0:23], dec_W1[46:47]], axis=0)
    w1h1 = jnp.concatenate([dec_W1[23:46], dec_W1[47:48]], axis=0)
    w1sq0 = dec_W1[50:51]
    w1sq1 = dec_W1[51:52]
    w1e = dec_W1[52:75]
    w1w = dec_W1[75:76]
    b1eff = (dec_b1 + dec_W1[48] + dec_W1[49]).reshape(1, -1)
    db2 = dec_b2.reshape(1, -1)
    db3 = dec_b3.reshape(1, -1)

    wr_spec = pl.BlockSpec((BLK, 1), lambda i: (jnp.minimum(i, NB - 1), 0))
    rd_spec = pl.BlockSpec((BLK, 1), lambda i: (jnp.maximum(i - NB, 0), 0))
    wspecs = [_full((1, 64)), _full((1, 64)), _full((64, 23)), _full((1, 23)),
              _full((23, 32)), _full((1, 32)), _full((32, 5)), _full((1, 5))]

    dec_pred = pl.pallas_call(
        _body,
        grid=(2 * NB,),
        in_specs=[wr_spec, wr_spec, rd_spec, rd_spec, *wspecs,
                  _full((1, 5)), _full((5, H * S)), _full((2 * 32, S)),
                  _full((24, 256)), _full((24, 256)),
                  _full((1, 256)), _full((1, 256)),
                  _full((23, 256)), _full((1, 256)), _full((1, 256)),
                  _full((256, 256)), _full((1, 256)),
                  _full((256, 1)), _full((1, 1))],
        out_specs=rd_spec,
        out_shape=jax.ShapeDtypeStruct((Q, 1), F32),
        scratch_shapes=[pltpu.VMEM((2 * 32, S), F32)],
    )(input_x, input_y, query_x, weight_sum_tensor,
      emb_W1.reshape(1, -1), eb1, emb_W2, eb2, ref_W1, rb1, ref_W2, rb2,
      amax, a_cat, minit,
      w1h0, w1h1, w1sq0, w1sq1, w1e, w1w, b1eff, dec_W2, db2, dec_W3, db3)

    return dec_pred
